# Initial kernel scaffold; baseline (speedup 1.0000x reference)
import jax
import jax.numpy as jnp
from jax.experimental import pallas as pl


def kernel(obs):
    raise NotImplementedError("write your pallas kernel here")


# speedup vs baseline: 9.5395x; 9.5395x over previous
import functools
import numpy as np
import jax
import jax.numpy as jnp
from jax import lax
from jax.experimental import pallas as pl
from jax.experimental.pallas import tpu as pltpu
from jax.experimental.pallas import tpu_sc as plsc

B = 16384
D_IN = 69
D_OUT = 27
NC = 2
NS = 16
NW = NC * NS
RW = B // NW
L = 16
NCHUNK = RW // L

_TP = float(np.float32(0.6366197723675814))
_BIG = float(np.float32(12582912.0))
_C1 = float(np.float32(np.round(np.pi / 2 * 2048.0) / 2048.0))
_C2 = float(np.float32(np.pi / 2 - np.float64(np.float32(np.round(np.pi / 2 * 2048.0) / 2048.0))))
_S1 = float(np.float32(-1.6666654611e-1))
_S2 = float(np.float32(8.3321608736e-3))
_S3 = float(np.float32(-1.9515295891e-4))
_CC1 = float(np.float32(4.166664568298827e-2))
_CC2 = float(np.float32(-1.388731625493765e-3))
_CC3 = float(np.float32(2.443315711809948e-5))


def _sincos(x):
    t = x * _TP
    n = (t + _BIG) - _BIG
    r = (x - n * _C1) - n * _C2
    ni = n.astype(jnp.int32)
    z = r * r
    s = r + r * z * (_S1 + z * (_S2 + z * _S3))
    c = 1.0 + z * (-0.5 + z * (_CC1 + z * (_CC2 + z * _CC3)))
    k = lax.bitwise_and(ni, 3)
    swap = lax.bitwise_and(k, 1) == 1
    sv = jnp.where(swap, c, s)
    cv = jnp.where(swap, s, c)
    sv = jnp.where(lax.bitwise_and(k, 2) == 2, -sv, sv)
    cv = jnp.where(lax.bitwise_and(k + 1, 2) == 2, -cv, cv)
    return sv, cv


def _mixer_body(obs_hbm, out_hbm, obs_v, out_v):
    cid = lax.axis_index("c")
    sid = lax.axis_index("s")
    wid = sid * NC + cid
    pltpu.sync_copy(obs_hbm.at[pl.ds(wid * RW * D_IN, RW * D_IN)], obs_v)

    iota = lax.iota(jnp.int32, L)

    def col(rbase, c):
        return plsc.load_gather(obs_v, [rbase + c])

    def putc(obase, cvec, x):
        plsc.store_scatter(out_v, [obase + cvec], x)

    def put(obase, c, x):
        plsc.store_scatter(out_v, [obase + c], x)

    def body(i, carry):
        rows = i * L + iota
        rbase = rows * D_IN
        obase = rows * D_OUT
        pos_x = col(rbase, 0)
        pos_y = col(rbase, 1)
        pos_z = col(rbase, 2)
        sa, ca = _sincos(col(rbase, 3))
        sb, cb = _sincos(col(rbase, 4))
        sc, cc = _sincos(col(rbase, 5))
        r00 = cb * ca
        r01 = -cb * sa
        r02 = sb
        scsb = sc * sb
        ccsb = cc * sb
        r10 = cc * sa + scsb * ca
        r11 = cc * ca - scsb * sa
        r12 = -sc * cb
        r20 = sc * sa - ccsb * ca
        r21 = sc * ca + ccsb * sa
        r22 = cc * cb

        def matvec(x, y, z):
            wx = pos_x - (r00 * x + r10 * y + r20 * z)
            wy = pos_y - (r01 * x + r11 * y + r21 * z)
            wz = pos_z - (r02 * x + r12 * y + r22 * z)
            return wx, wy, wz

        px = []
        py = []
        pz = []
        c2 = []
        for k in range(7):
            x = col(rbase, 13 + 7 * k)
            y = col(rbase, 14 + 7 * k)
            z = col(rbase, 15 + 7 * k)
            px.append(x)
            py.append(y)
            pz.append(z)
            c2.append(x * x + y * y + z * z)
        zero = jnp.zeros((L,), jnp.int32)
        rk = [zero] * 7
        one = jnp.full((L,), 1, jnp.int32)
        for j in range(7):
            for k in range(j + 1, 7):
                m = c2[j] <= c2[k]
                rk[k] = rk[k] + jnp.where(m, one, zero)
                rk[j] = rk[j] + jnp.where(m, zero, one)

        put(obase, 0, pos_x)
        put(obase, 1, pos_y)
        put(obase, 2, pos_z)
        for k in range(7):
            wx, wy, wz = matvec(px[k], py[k], pz[k])
            cbase = 3 + 3 * rk[k]
            putc(obase, cbase, wx)
            putc(obase, cbase + 1, wy)
            putc(obase, cbase + 2, wz)
        wx, wy, wz = matvec(col(rbase, 62), col(rbase, 63), col(rbase, 64))
        put(obase, 24, wx)
        put(obase, 25, wy)
        put(obase, 26, wz)
        return carry

    lax.fori_loop(0, NCHUNK, body, 0)
    pltpu.sync_copy(out_v, out_hbm.at[pl.ds(wid * RW * D_OUT, RW * D_OUT)])


@jax.jit
def kernel(obs):
    mesh = plsc.VectorSubcoreMesh(core_axis_name="c", subcore_axis_name="s")
    f = functools.partial(
        pl.kernel,
        mesh=mesh,
        out_type=jax.ShapeDtypeStruct((B * D_OUT,), jnp.float32),
        scratch_types=[
            pltpu.VMEM((RW * D_IN,), jnp.float32),
            pltpu.VMEM((RW * D_OUT,), jnp.float32),
        ],
        compiler_params=pltpu.CompilerParams(needs_layout_passes=False),
    )(_mixer_body)
    return f(obs.reshape(-1)).reshape(B, D_OUT)
